# baseline (speedup 1.0000x reference)
import jax
import jax.numpy as jnp
from jax import lax
from jax.experimental import pallas as pl
from jax.experimental.pallas import tpu as pltpu
from jax.experimental.pallas import tpu_sc as plsc

BLK_S = 1024
_NPAD = 8


def _sc_gather_body(table_hbm, idx_hbm, emb_hbm, idx_v, rows_v, sem):
    s = lax.axis_index("s")

    @pl.when(s == 0)
    def _():
        pltpu.sync_copy(idx_hbm, idx_v)
        pltpu.async_copy(table_hbm.at[idx_v], rows_v, sem).wait()
        pltpu.sync_copy(rows_v, emb_hbm)


def _sc_gather(table, idx_pad):
    V, D = table.shape
    mesh = plsc.VectorSubcoreMesh(
        core_axis_name="c", subcore_axis_name="s", num_cores=1
    )
    run = pl.kernel(
        _sc_gather_body,
        mesh=mesh,
        out_type=jax.ShapeDtypeStruct((_NPAD, D), jnp.float32),
        scratch_types=[
            pltpu.VMEM((_NPAD,), jnp.int32),
            pltpu.VMEM((_NPAD, D), jnp.float32),
            pltpu.SemaphoreType.DMA,
        ],
    )
    return run(table, idx_pad)


def _add_body(x_ref, emb_ref, o_ref):
    o_ref[...] = x_ref[...] + emb_ref[0]


def kernel(x, tissue_vector, registry_tokens):
    B, S, D = x.shape
    idx = tissue_vector[:, 0].astype(jnp.int32)
    idx_pad = jnp.zeros((_NPAD,), jnp.int32).at[:B].set(idx)
    emb = _sc_gather(registry_tokens, idx_pad)
    emb3 = emb.reshape(_NPAD, 1, D)
    x2 = x.reshape(B * S, D)
    blks_per_b = S // BLK_S
    grid = (B * blks_per_b,)
    out = pl.pallas_call(
        _add_body,
        grid=grid,
        in_specs=[
            pl.BlockSpec((BLK_S, D), lambda i: (i, 0)),
            pl.BlockSpec((1, 1, D), lambda i: (i // blks_per_b, 0, 0)),
        ],
        out_specs=pl.BlockSpec((BLK_S, D), lambda i: (i, 0)),
        out_shape=jax.ShapeDtypeStruct((B * S, D), x.dtype),
        compiler_params=pltpu.CompilerParams(
            dimension_semantics=("arbitrary",),
        ),
    )(x2, emb3)
    return out.reshape(B, S, D)
